# baseline (speedup 1.0000x reference)
import jax
import jax.numpy as jnp
from jax.experimental import pallas as pl

_Q = 1000
_K = 4
_BLK = 256


def _body(q_ref, r_ref, o_ref):
    q = q_ref[...]
    r = r_ref[...]
    col = jax.lax.broadcasted_iota(jnp.int32, (_BLK, _K * _Q), 1)
    k = col // _Q
    qpos = col - k * _Q
    dist = jnp.abs(k - r).astype(jnp.float32) * (1.0 / (_K - 1))
    w = jnp.maximum(1.0 - dist, 0.0)
    o_ref[...] = jnp.where(qpos == q - 1, w, 0.0)


def kernel(question_ids, responses):
    B, S = responses.shape
    n = B * S
    q2 = question_ids.reshape(n, 1).astype(jnp.int32)
    r2 = responses.reshape(n, 1).astype(jnp.int32)
    out = pl.pallas_call(
        _body,
        out_shape=jax.ShapeDtypeStruct((n, _K * _Q), jnp.float32),
        grid=(n // _BLK,),
        in_specs=[
            pl.BlockSpec((_BLK, 1), lambda i: (i, 0)),
            pl.BlockSpec((_BLK, 1), lambda i: (i, 0)),
        ],
        out_specs=pl.BlockSpec((_BLK, _K * _Q), lambda i: (i, 0)),
    )(q2, r2)
    return out.reshape(B, S, _K * _Q)


# speedup vs baseline: 1.0408x; 1.0408x over previous
import jax
import jax.numpy as jnp
from jax.experimental import pallas as pl

_Q = 1000
_K = 4
_BLK = 256


def _body(qm1_ref, r3_ref, qpos_ref, kf3_ref, o_ref):
    qm1 = qm1_ref[...]
    r3 = r3_ref[...]
    qpos = qpos_ref[...]
    kf3 = kf3_ref[...]
    w = 1.0 - jnp.abs(kf3 - r3)
    o_ref[...] = jnp.where(qpos == qm1, w, 0.0)


def kernel(question_ids, responses):
    B, S = responses.shape
    n = B * S
    qm1 = question_ids.reshape(n, 1).astype(jnp.int32) - 1
    r3 = responses.reshape(n, 1).astype(jnp.float32) * (1.0 / (_K - 1))
    col = jnp.arange(_K * _Q, dtype=jnp.int32)
    qpos = (col % _Q).reshape(1, _K * _Q)
    kf3 = ((col // _Q).astype(jnp.float32) * (1.0 / (_K - 1))).reshape(1, _K * _Q)
    out = pl.pallas_call(
        _body,
        out_shape=jax.ShapeDtypeStruct((n, _K * _Q), jnp.float32),
        grid=(n // _BLK,),
        in_specs=[
            pl.BlockSpec((_BLK, 1), lambda i: (i, 0)),
            pl.BlockSpec((_BLK, 1), lambda i: (i, 0)),
            pl.BlockSpec((1, _K * _Q), lambda i: (0, 0)),
            pl.BlockSpec((1, _K * _Q), lambda i: (0, 0)),
        ],
        out_specs=pl.BlockSpec((_BLK, _K * _Q), lambda i: (i, 0)),
    )(qm1, r3, qpos, kf3)
    return out.reshape(B, S, _K * _Q)


# speedup vs baseline: 1.5417x; 1.4813x over previous
import jax
import jax.numpy as jnp
from jax.experimental import pallas as pl

_Q = 1000
_K = 4
_BB = 16


def _body(qm1_ref, r3_ref, qpos_ref, kf3_ref, o_ref):
    qm1 = qm1_ref[...][:, :, None]
    r3 = r3_ref[...][:, :, None]
    qpos = qpos_ref[...]
    kf3 = kf3_ref[...]
    w = 1.0 - jnp.abs(kf3 - r3)
    o_ref[...] = jnp.where(qpos == qm1, w, 0.0)


def kernel(question_ids, responses):
    B, S = responses.shape
    qm1 = question_ids.astype(jnp.int32) - 1
    r3 = responses.astype(jnp.float32) * (1.0 / (_K - 1))
    col = jnp.arange(_K * _Q, dtype=jnp.int32)
    qpos = (col % _Q).reshape(1, 1, _K * _Q)
    kf3 = ((col // _Q).astype(jnp.float32) * (1.0 / (_K - 1))).reshape(1, 1, _K * _Q)
    return pl.pallas_call(
        _body,
        out_shape=jax.ShapeDtypeStruct((B, S, _K * _Q), jnp.float32),
        grid=(B // _BB,),
        in_specs=[
            pl.BlockSpec((_BB, S), lambda i: (i, 0)),
            pl.BlockSpec((_BB, S), lambda i: (i, 0)),
            pl.BlockSpec((1, 1, _K * _Q), lambda i: (0, 0, 0)),
            pl.BlockSpec((1, 1, _K * _Q), lambda i: (0, 0, 0)),
        ],
        out_specs=pl.BlockSpec((_BB, S, _K * _Q), lambda i: (i, 0, 0)),
    )(qm1, r3, qpos, kf3)


# speedup vs baseline: 1.5694x; 1.0180x over previous
import jax
import jax.numpy as jnp
from jax import lax
from jax.experimental import pallas as pl
from jax.experimental.pallas import tpu as pltpu

_Q = 1000
_K = 4
_BB = 8
_NBUF = 4


def _body(qm1_ref, r3_ref, qpos_ref, kf3_ref, o_hbm, buf, sems):
    B = qm1_ref.shape[0]
    qpos = qpos_ref[...]
    kf3 = kf3_ref[...]
    nsteps = B // _BB

    def compute(g, b):
        qm1 = qm1_ref[pl.ds(g * _BB, _BB), :][:, :, None]
        r3 = r3_ref[pl.ds(g * _BB, _BB), :][:, :, None]
        w = 1.0 - jnp.abs(kf3 - r3)
        buf[b] = jnp.where(qpos == qm1, w, 0.0)

    def dma(g, b):
        return pltpu.make_async_copy(
            buf.at[b], o_hbm.at[pl.ds(g * _BB, _BB)], sems.at[b])

    def step(g, carry):
        b = lax.rem(g, _NBUF)

        @pl.when(g >= _NBUF)
        def _():
            dma(g - _NBUF, b).wait()

        compute(g, b)
        dma(g, b).start()
        return carry

    lax.fori_loop(0, nsteps, step, 0)

    def drain(i, carry):
        g = nsteps - _NBUF + i
        dma(g, lax.rem(g, _NBUF)).wait()
        return carry

    lax.fori_loop(0, _NBUF, drain, 0)


def kernel(question_ids, responses):
    B, S = responses.shape
    qm1 = question_ids.astype(jnp.int32) - 1
    r3 = responses.astype(jnp.float32) * (1.0 / (_K - 1))
    col = jnp.arange(_K * _Q, dtype=jnp.int32)
    qpos = (col % _Q).reshape(1, 1, _K * _Q)
    kf3 = ((col // _Q).astype(jnp.float32) * (1.0 / (_K - 1))).reshape(1, 1, _K * _Q)
    return pl.pallas_call(
        _body,
        out_shape=jax.ShapeDtypeStruct((B, S, _K * _Q), jnp.float32),
        in_specs=[
            pl.BlockSpec(memory_space=pltpu.VMEM),
            pl.BlockSpec(memory_space=pltpu.VMEM),
            pl.BlockSpec(memory_space=pltpu.VMEM),
            pl.BlockSpec(memory_space=pltpu.VMEM),
        ],
        out_specs=pl.BlockSpec(memory_space=pl.ANY),
        scratch_shapes=[
            pltpu.VMEM((_NBUF, _BB, S, _K * _Q), jnp.float32),
            pltpu.SemaphoreType.DMA((_NBUF,)),
        ],
    )(qm1, r3, qpos, kf3)
